# baseline (speedup 1.0000x reference)
import functools

import jax
import jax.numpy as jnp
from jax import lax
from jax.experimental import pallas as pl
from jax.experimental.pallas import tpu as pltpu
from jax.experimental.pallas import tpu_sc as plsc

B = 4
C = 8
T = 4096
V = 2048
D = 1024

NROWS = B * T
NW = 32
RPW = NROWS // NW
R = 8
NCH = RPW // R
G = R * C
NL = 16
DW = D // 2


def _sc_embed(codes_flat, tables_packed):
    mesh = plsc.VectorSubcoreMesh(core_axis_name="c", subcore_axis_name="s")

    @functools.partial(
        pl.kernel,
        mesh=mesh,
        out_type=jax.ShapeDtypeStruct((NROWS, DW), jnp.int32),
        scratch_types=[
            pltpu.VMEM((G,), jnp.int32),
            pltpu.VMEM((G, DW), jnp.int32),
            pltpu.VMEM((R, DW), jnp.int32),
            pltpu.SemaphoreType.DMA,
        ],
    )
    def k(codes_hbm, tab_hbm, out_hbm, cbuf, gbuf, obuf, sem):
        wid = lax.axis_index("s") * 2 + lax.axis_index("c")
        base = wid * RPW
        lane = lax.iota(jnp.int32, NL)
        offpat = (lane & (C - 1)) * V
        himask = jnp.full((NL,), -65536, jnp.int32)
        half = jnp.full((NL,), 0x8000, jnp.int32)

        def chunk(ci, _):
            row0 = base + ci * R
            pltpu.sync_copy(codes_hbm.at[pl.ds(row0 * C, G)], cbuf)
            for g in range(G // NL):
                sl = pl.ds(g * NL, NL)
                cbuf[sl] = cbuf[sl] + offpat
            pltpu.async_copy(tab_hbm.at[cbuf], gbuf, sem).wait()

            def reduce_group(g, _):
                sl = pl.ds(g * NL, NL)
                for r in range(R):
                    w = gbuf[r * C, sl]
                    lo = lax.bitcast_convert_type(w << 16, jnp.float32)
                    hi = lax.bitcast_convert_type(w & himask, jnp.float32)
                    for i in range(1, C):
                        w = gbuf[r * C + i, sl]
                        lo = lo + lax.bitcast_convert_type(w << 16,
                                                           jnp.float32)
                        hi = hi + lax.bitcast_convert_type(w & himask,
                                                           jnp.float32)
                    lob = lax.bitcast_convert_type(lo, jnp.int32) + half
                    hib = lax.bitcast_convert_type(hi, jnp.int32) + half
                    obuf[r, sl] = (hib & himask) | lax.shift_right_logical(
                        lob, 16)
                return 0

            lax.fori_loop(0, DW // NL, reduce_group, 0)
            pltpu.sync_copy(obuf, out_hbm.at[pl.ds(row0, R)])
            return 0

        lax.fori_loop(0, NCH, chunk, 0)

    return k(codes_flat, tables_packed)


def kernel(codes, tables):
    codes_flat = codes.transpose(0, 2, 1).reshape(-1)
    tables_packed = jax.lax.bitcast_convert_type(
        tables.astype(jnp.bfloat16).reshape(C * V, DW, 2), jnp.int32)
    out = _sc_embed(codes_flat, tables_packed)
    out_bf = jax.lax.bitcast_convert_type(out, jnp.bfloat16)
    return out_bf.astype(jnp.float32).reshape(B, T, D)


# speedup vs baseline: 1.3303x; 1.3303x over previous
import functools

import jax
import jax.numpy as jnp
from jax import lax
from jax.experimental import pallas as pl
from jax.experimental.pallas import tpu as pltpu
from jax.experimental.pallas import tpu_sc as plsc

B = 4
C = 8
T = 4096
V = 2048
D = 1024

NROWS = B * T
NW = 32
WPB = NW // B
RPW = NROWS // NW
R = 8
NCH = RPW // R
NL = 16
DW = D // 2


def _sc_embed(codes_nat, tables_packed):
    mesh = plsc.VectorSubcoreMesh(core_axis_name="c", subcore_axis_name="s")

    @functools.partial(
        pl.kernel,
        mesh=mesh,
        out_type=jax.ShapeDtypeStruct((NROWS, D), jnp.float32),
        scratch_types=[
            pltpu.VMEM((C, RPW), jnp.int32),
            pltpu.VMEM((C * R, DW), jnp.int32),
            pltpu.VMEM((R, D), jnp.float32),
            pltpu.SemaphoreType.DMA,
        ],
    )
    def k(codes_hbm, tab_hbm, out_hbm, cvmem, gbuf, obuf, sem):
        wid = lax.axis_index("s") * 2 + lax.axis_index("c")
        base = wid * RPW
        bi = wid // WPB
        t0 = (wid % WPB) * RPW
        himask = jnp.full((NL,), -65536, jnp.int32)

        for i in range(C):
            pltpu.sync_copy(codes_hbm.at[bi, i, pl.ds(t0, RPW)], cvmem.at[i])

        def add_off(g, _):
            sl = pl.ds(g * NL, NL)
            for i in range(1, C):
                cvmem[i, sl] = cvmem[i, sl] + (i * V)
            return 0

        lax.fori_loop(0, RPW // NL, add_off, 0)

        def chunk(ci, _):
            for i in range(C):
                pltpu.async_copy(
                    tab_hbm.at[cvmem.at[i, pl.ds(ci * R, R)]],
                    gbuf.at[pl.ds(i * R, R)], sem)
            for i in range(C):
                pltpu.make_async_copy(
                    tab_hbm.at[cvmem.at[i, pl.ds(ci * R, R)]],
                    gbuf.at[pl.ds(i * R, R)], sem).wait()

            def reduce_group(g, _):
                sl = pl.ds(g * NL, NL)
                for r in range(R):
                    w = gbuf[r, sl]
                    lo = lax.bitcast_convert_type(w << 16, jnp.float32)
                    hi = lax.bitcast_convert_type(w & himask, jnp.float32)
                    for i in range(1, C):
                        w = gbuf[i * R + r, sl]
                        lo = lo + lax.bitcast_convert_type(w << 16,
                                                           jnp.float32)
                        hi = hi + lax.bitcast_convert_type(w, jnp.float32)
                    obuf[r, sl] = lo
                    obuf[r, pl.ds(DW + g * NL, NL)] = hi
                return 0

            lax.fori_loop(0, DW // NL, reduce_group, 0)
            pltpu.sync_copy(obuf, out_hbm.at[pl.ds(base + ci * R, R)])
            return 0

        lax.fori_loop(0, NCH, chunk, 0)

    return k(codes_nat, tables_packed)


def kernel(codes, tables):
    tbf = tables.astype(jnp.bfloat16).reshape(C * V, 2, DW)
    tables_packed = jax.lax.bitcast_convert_type(
        jnp.stack([tbf[:, 0, :], tbf[:, 1, :]], axis=-1), jnp.int32)
    out = _sc_embed(codes, tables_packed)
    return out.reshape(B, T, D)


# speedup vs baseline: 1.7738x; 1.3334x over previous
import functools

import jax
import jax.numpy as jnp
from jax import lax
from jax.experimental import pallas as pl
from jax.experimental.pallas import tpu as pltpu
from jax.experimental.pallas import tpu_sc as plsc

B = 4
C = 8
T = 4096
V = 2048
D = 1024

NROWS = B * T
NW = 32
WPB = NW // B
RPW = NROWS // NW
R = 8
NCH = RPW // R
NL = 16
DW = D // 2


def _sc_embed(codes_nat, tables_packed):
    mesh = plsc.VectorSubcoreMesh(core_axis_name="c", subcore_axis_name="s")

    @functools.partial(
        pl.kernel,
        mesh=mesh,
        out_type=jax.ShapeDtypeStruct((NROWS, D), jnp.float32),
        scratch_types=[
            pltpu.VMEM((C, RPW), jnp.int32),
            pltpu.VMEM((C * R, DW), jnp.int32),
            pltpu.VMEM((R, D), jnp.float32),
            pltpu.SemaphoreType.DMA,
        ],
    )
    def k(codes_hbm, tab_hbm, out_hbm, cvmem, gbuf, obuf, sem):
        wid = lax.axis_index("s") * 2 + lax.axis_index("c")
        base = wid * RPW
        bi = wid // WPB
        t0 = (wid % WPB) * RPW
        himask = jnp.full((NL,), -65536, jnp.int32)

        for i in range(C):
            pltpu.sync_copy(codes_hbm.at[bi, i, pl.ds(t0, RPW)], cvmem.at[i])

        def add_off(g, _):
            sl = pl.ds(g * NL, NL)
            for i in range(1, C):
                cvmem[i, sl] = cvmem[i, sl] + (i * V)
            return 0

        lax.fori_loop(0, RPW // NL, add_off, 0)

        def chunk(ci, _):
            for i in range(C):
                pltpu.async_copy(
                    tab_hbm.at[cvmem.at[i, pl.ds(ci * R, R)]],
                    gbuf.at[pl.ds(i * R, R)], sem)
            for i in range(C):
                pltpu.make_async_copy(
                    tab_hbm.at[cvmem.at[i, pl.ds(ci * R, R)]],
                    gbuf.at[pl.ds(i * R, R)], sem).wait()

            def reduce_group(g, _):
                sl = pl.ds(g * NL, NL)
                for r in range(R):
                    w = gbuf[r, sl]
                    lo = lax.bitcast_convert_type(w << 16, jnp.float32)
                    hi = lax.bitcast_convert_type(w & himask, jnp.float32)
                    for i in range(1, C):
                        w = gbuf[i * R + r, sl]
                        lo = lo + lax.bitcast_convert_type(w << 16,
                                                           jnp.float32)
                        hi = hi + lax.bitcast_convert_type(w, jnp.float32)
                    obuf[r, sl] = lo
                    obuf[r, pl.ds(DW + g * NL, NL)] = hi
                return 0

            lax.fori_loop(0, DW // NL, reduce_group, 0)
            pltpu.sync_copy(obuf, out_hbm.at[pl.ds(base + ci * R, R)])
            return 0

        lax.fori_loop(0, NCH, chunk, 0)

    return k(codes_nat, tables_packed)


def kernel(codes, tables):
    tu = jax.lax.bitcast_convert_type(
        tables.astype(jnp.bfloat16), jnp.uint16).reshape(C * V, 2, DW)
    tables_packed = jax.lax.bitcast_convert_type(
        tu[:, 0, :].astype(jnp.uint32)
        | (tu[:, 1, :].astype(jnp.uint32) << 16), jnp.int32)
    out = _sc_embed(codes, tables_packed)
    return out.reshape(B, T, D)


# speedup vs baseline: 2.4236x; 1.3664x over previous
import functools

import jax
import jax.numpy as jnp
from jax import lax
from jax.experimental import pallas as pl
from jax.experimental.pallas import tpu as pltpu
from jax.experimental.pallas import tpu_sc as plsc

B = 4
C = 8
T = 4096
V = 2048
D = 1024

NROWS = B * T
NW = 32
WPB = NW // B
RPW = NROWS // NW
R = 8
NCH = RPW // R
NL = 16
DW = D // 2


def _sc_embed(codes_nat, tables_packed):
    mesh = plsc.VectorSubcoreMesh(core_axis_name="c", subcore_axis_name="s")

    @functools.partial(
        pl.kernel,
        mesh=mesh,
        out_type=jax.ShapeDtypeStruct((NROWS, D), jnp.float32),
        scratch_types=[
            pltpu.VMEM((C, RPW), jnp.int32),
            pltpu.VMEM((C * R, DW), jnp.int32),
            pltpu.VMEM((R, D), jnp.float32),
            pltpu.SemaphoreType.DMA,
        ],
    )
    def k(codes_hbm, tab_hbm, out_hbm, cvmem, gbuf, obuf, sem):
        wid = lax.axis_index("s") * 2 + lax.axis_index("c")
        base = wid * RPW
        bi = wid // WPB
        t0 = (wid % WPB) * RPW
        himask = jnp.full((NL,), -65536, jnp.int32)

        for i in range(C):
            pltpu.sync_copy(codes_hbm.at[bi, i, pl.ds(t0, RPW)], cvmem.at[i])

        def add_off(g, _):
            sl = pl.ds(g * NL, NL)
            for i in range(1, C):
                cvmem[i, sl] = cvmem[i, sl] + (i * V)
            return 0

        lax.fori_loop(0, RPW // NL, add_off, 0)

        def chunk(ci, _):
            for i in range(C):
                pltpu.async_copy(
                    tab_hbm.at[cvmem.at[i, pl.ds(ci * R, R)]],
                    gbuf.at[pl.ds(i * R, R)], sem)
            for i in range(C):
                pltpu.make_async_copy(
                    tab_hbm.at[cvmem.at[i, pl.ds(ci * R, R)]],
                    gbuf.at[pl.ds(i * R, R)], sem).wait()

            def reduce_group(g, _):
                sl = pl.ds(g * NL, NL)
                for r in range(R):
                    w = gbuf[r, sl]
                    lo = lax.bitcast_convert_type(w << 16, jnp.float32)
                    hi = lax.bitcast_convert_type(w & himask, jnp.float32)
                    for i in range(1, C):
                        w = gbuf[i * R + r, sl]
                        lo = lo + lax.bitcast_convert_type(w << 16,
                                                           jnp.float32)
                        hi = hi + lax.bitcast_convert_type(w, jnp.float32)
                    obuf[r, sl] = lo
                    obuf[r, pl.ds(DW + g * NL, NL)] = hi
                return 0

            lax.fori_loop(0, DW // NL, reduce_group, 0)
            pltpu.sync_copy(obuf, out_hbm.at[pl.ds(base + ci * R, R)])
            return 0

        lax.fori_loop(0, NCH, chunk, 0)

    return k(codes_nat, tables_packed)


def kernel(codes, tables):
    ti = jax.lax.bitcast_convert_type(tables, jnp.int32).reshape(C * V, D)
    a = ti[:, :DW] + 0x8000
    b = ti[:, DW:] + 0x8000
    tables_packed = (jax.lax.shift_right_logical(a, 16) | (b & -65536))
    out = _sc_embed(codes, tables_packed)
    return out.reshape(B, T, D)


# speedup vs baseline: 3.5353x; 1.4586x over previous
import functools

import jax
import jax.numpy as jnp
from jax import lax
from jax.experimental import pallas as pl
from jax.experimental.pallas import tpu as pltpu
from jax.experimental.pallas import tpu_sc as plsc

B = 4
C = 8
T = 4096
V = 2048
D = 1024

NROWS = B * T
NW = 32
WPB = NW // B
RPW = NROWS // NW
R = 8
NCH = RPW // R
NL = 16
DW = D // 2


def _sc_embed(codes_nat, tables_packed):
    mesh = plsc.VectorSubcoreMesh(core_axis_name="c", subcore_axis_name="s")

    @functools.partial(
        pl.kernel,
        mesh=mesh,
        out_type=jax.ShapeDtypeStruct((NROWS, D), jnp.float32),
        scratch_types=[
            pltpu.VMEM((C, RPW), jnp.int32),
            pltpu.VMEM((C * R, DW), jnp.int32),
            pltpu.VMEM((C * R, DW), jnp.int32),
            pltpu.VMEM((R, D), jnp.float32),
            pltpu.SemaphoreType.DMA,
            pltpu.SemaphoreType.DMA,
        ],
    )
    def k(codes_hbm, tab_hbm, out_hbm, cvmem, gbuf0, gbuf1, obuf,
          sem0, sem1):
        gbufs = (gbuf0, gbuf1)
        sems = (sem0, sem1)
        wid = lax.axis_index("s") * 2 + lax.axis_index("c")
        base = wid * RPW
        bi = wid // WPB
        t0 = (wid % WPB) * RPW
        himask = jnp.full((NL,), -65536, jnp.int32)

        for i in range(C):
            pltpu.sync_copy(codes_hbm.at[bi, i, pl.ds(t0, RPW)], cvmem.at[i])

        def add_off(g, _):
            sl = pl.ds(g * NL, NL)
            for i in range(1, C):
                cvmem[i, sl] = cvmem[i, sl] + (i * V)
            return 0

        lax.fori_loop(0, RPW // NL, add_off, 0)

        def issue(ci, gbuf, sem):
            for i in range(C):
                pltpu.async_copy(
                    tab_hbm.at[cvmem.at[i, pl.ds(ci * R, R)]],
                    gbuf.at[pl.ds(i * R, R)], sem)

        def drain(ci, gbuf, sem):
            for i in range(C):
                pltpu.make_async_copy(
                    tab_hbm.at[cvmem.at[i, pl.ds(ci * R, R)]],
                    gbuf.at[pl.ds(i * R, R)], sem).wait()

        issue(0, gbuf0, sem0)

        def pair(p, _):
            for b in range(2):
                ci = p * 2 + b
                gbuf = gbufs[b]

                @pl.when(ci + 1 < NCH)
                def _():
                    issue(ci + 1, gbufs[1 - b], sems[1 - b])

                drain(ci, gbuf, sems[b])

                def reduce_group(g, _):
                    sl = pl.ds(g * NL, NL)
                    for r in range(R):
                        w = gbuf[r, sl]
                        lo = lax.bitcast_convert_type(w << 16, jnp.float32)
                        hi = lax.bitcast_convert_type(w & himask, jnp.float32)
                        for i in range(1, C):
                            w = gbuf[i * R + r, sl]
                            lo = lo + lax.bitcast_convert_type(w << 16,
                                                               jnp.float32)
                            hi = hi + lax.bitcast_convert_type(w, jnp.float32)
                        obuf[r, sl] = lo
                        obuf[r, pl.ds(DW + g * NL, NL)] = hi
                    return 0

                lax.fori_loop(0, DW // NL, reduce_group, 0)
                pltpu.sync_copy(obuf, out_hbm.at[pl.ds(base + ci * R, R)])
            return 0

        lax.fori_loop(0, NCH // 2, pair, 0)

    return k(codes_nat, tables_packed)


def kernel(codes, tables):
    ti = jax.lax.bitcast_convert_type(tables, jnp.int32).reshape(C * V, D)
    a = ti[:, :DW] + 0x8000
    b = ti[:, DW:] + 0x8000
    tables_packed = (jax.lax.shift_right_logical(a, 16) | (b & -65536))
    out = _sc_embed(codes, tables_packed)
    return out.reshape(B, T, D)


# speedup vs baseline: 4.1825x; 1.1831x over previous
import functools

import jax
import jax.numpy as jnp
from jax import lax
from jax.experimental import pallas as pl
from jax.experimental.pallas import tpu as pltpu
from jax.experimental.pallas import tpu_sc as plsc

B = 4
C = 8
T = 4096
V = 2048
D = 1024

NROWS = B * T
NW = 32
WPB = NW // B
RPW = NROWS // NW
R = 8
NCH = RPW // R
NL = 16
DW = D // 2


def _sc_embed(codes_nat, tables_packed):
    mesh = plsc.VectorSubcoreMesh(core_axis_name="c", subcore_axis_name="s")

    @functools.partial(
        pl.kernel,
        mesh=mesh,
        out_type=jax.ShapeDtypeStruct((NROWS, D), jnp.float32),
        scratch_types=[
            pltpu.VMEM((C, RPW), jnp.int32),
            pltpu.VMEM((C * R, DW), jnp.int32),
            pltpu.VMEM((C * R, DW), jnp.int32),
            pltpu.VMEM((R, D), jnp.float32),
            pltpu.SemaphoreType.DMA,
            pltpu.SemaphoreType.DMA,
        ],
    )
    def k(codes_hbm, tab_hbm, out_hbm, cvmem, gbuf0, gbuf1, obuf,
          sem0, sem1):
        gbufs = (gbuf0, gbuf1)
        sems = (sem0, sem1)
        wid = lax.axis_index("s") * 2 + lax.axis_index("c")
        base = wid * RPW
        bi = wid // WPB
        t0 = (wid % WPB) * RPW
        himask = jnp.full((NL,), -65536, jnp.int32)

        for i in range(C):
            pltpu.sync_copy(codes_hbm.at[bi, i, pl.ds(t0, RPW)], cvmem.at[i])

        def add_off(g, _):
            sl = pl.ds(g * NL, NL)
            for i in range(1, C):
                cvmem[i, sl] = cvmem[i, sl] + (i * V)
            return 0

        lax.fori_loop(0, RPW // NL, add_off, 0)

        def issue(ci, gbuf, sem):
            for i in range(C):
                pltpu.async_copy(
                    tab_hbm.at[cvmem.at[i, pl.ds(ci * R, R)]],
                    gbuf.at[pl.ds(i * R, R)], sem)

        def drain(ci, gbuf, sem):
            for i in range(C):
                pltpu.make_async_copy(
                    tab_hbm.at[cvmem.at[i, pl.ds(ci * R, R)]],
                    gbuf.at[pl.ds(i * R, R)], sem).wait()

        issue(0, gbuf0, sem0)

        def pair(p, _):
            for b in range(2):
                ci = p * 2 + b
                gbuf = gbufs[b]

                @pl.when(ci + 1 < NCH)
                def _():
                    issue(ci + 1, gbufs[1 - b], sems[1 - b])

                drain(ci, gbuf, sems[b])

                def reduce_group(g, _):
                    sl = pl.ds(g * NL, NL)
                    for r in range(R):
                        w = gbuf[r, sl]
                        lo = lax.bitcast_convert_type(w << 16, jnp.float32)
                        hi = lax.bitcast_convert_type(w & himask, jnp.float32)
                        for i in range(1, C):
                            w = gbuf[i * R + r, sl]
                            lo = lo + lax.bitcast_convert_type(w << 16,
                                                               jnp.float32)
                            hi = hi + lax.bitcast_convert_type(w, jnp.float32)
                        obuf[r, sl] = lo
                        obuf[r, pl.ds(DW + g * NL, NL)] = hi
                    return 0

                lax.fori_loop(0, DW // NL, reduce_group, 0)
                pltpu.sync_copy(obuf, out_hbm.at[pl.ds(base + ci * R, R)])
            return 0

        lax.fori_loop(0, NCH // 2, pair, 0)

    return k(codes_nat, tables_packed)


def kernel(codes, tables):
    tf = tables.reshape(C * V, D)
    a = jax.lax.bitcast_convert_type(tf[:, :DW], jnp.int32) + 0x8000
    b = jax.lax.bitcast_convert_type(tf[:, DW:], jnp.int32) + 0x8000
    tables_packed = (jax.lax.shift_right_logical(a, 16) | (b & -65536))
    out = _sc_embed(codes, tables_packed)
    return out.reshape(B, T, D)
